# baseline (speedup 1.0000x reference)
import functools

import jax
import jax.numpy as jnp
from jax import lax
from jax.experimental import pallas as pl
from jax.experimental.pallas import tpu as pltpu
from jax.experimental.pallas import tpu_sc as plsc

CONF_THRES = 0.1
IOU_THRES = 0.6
MAX_DET = 100
IMG_SIZE = 512.0

_N = 5000
_NP = 5120
_R, _C = 8, 640
_DET = 128

_LEVELS = (
    (64, 64, 192, 1.0 / 8, 0),
    (32, 32, 384, 1.0 / 16, 192),
    (16, 16, 576, 1.0 / 32, 576),
    (8, 8, 768, 1.0 / 64, 1152),
)


def _nms_body(sc_ref, x1_ref, y1_ref, x2_ref, y2_ref, bsm_ref,
              keep_ref, idx_ref, w_ref, bv_ref):
    s0 = jnp.where(sc_ref[:, :] > CONF_THRES, sc_ref[:, :], -1.0)
    lane_iota = lax.broadcasted_iota(jnp.int32, (1, _DET), 1)

    def step(t, carry):
        s, keep_row, vx1, vy1, vx2, vy2, vvalid = carry
        x1 = x1_ref[:, :]
        y1 = y1_ref[:, :]
        x2 = x2_ref[:, :]
        y2 = y2_ref[:, :]
        ridx = lax.broadcasted_iota(jnp.int32, (_R, _C), 0)
        cidx = lax.broadcasted_iota(jnp.int32, (_R, _C), 1)
        gidx = ridx * _C + cidx
        rowmax = jnp.max(s, axis=1, keepdims=True)
        rowarg = jnp.argmax(s, axis=1, keepdims=True)
        riota = lax.broadcasted_iota(jnp.int32, (_R, 1), 0)
        best = jnp.max(rowmax, axis=0, keepdims=True)
        rhit = rowmax == best
        r = jnp.min(jnp.where(rhit, riota, _R), axis=0, keepdims=True)
        carg = jnp.min(jnp.where(riota == r, rowarg, _C),
                       axis=0, keepdims=True)
        idxv = r * _C + carg
        idx_s = idxv[0, 0]
        base = idx_s * 4
        bx1 = bsm_ref[base]
        by1 = bsm_ref[base + 1]
        bx2 = bsm_ref[base + 2]
        by2 = bsm_ref[base + 3]
        barea = (bx2 - bx1) * (by2 - by1)
        area = (x2 - x1) * (y2 - y1)
        xx1 = jnp.maximum(bx1, x1)
        yy1 = jnp.maximum(by1, y1)
        xx2 = jnp.minimum(bx2, x2)
        yy2 = jnp.minimum(by2, y2)
        inter = jnp.maximum(xx2 - xx1, 0.0) * jnp.maximum(yy2 - yy1, 0.0)
        iou = inter / (barea + area - inter + 1e-9)
        pos = best > 0.0
        suppress = ((iou > IOU_THRES) | (gidx == idx_s)) & pos
        s_new = jnp.where(suppress, -1.0, s)
        pos0 = pos[0:1, 0:1]
        out_idx = jnp.where(pos0, idx_s, -1)
        lane_m = lane_iota == t
        keep_row = jnp.where(lane_m, out_idx, keep_row)
        vx1 = jnp.where(lane_m, bx1, vx1)
        vy1 = jnp.where(lane_m, by1, vy1)
        vx2 = jnp.where(lane_m, bx2, vx2)
        vy2 = jnp.where(lane_m, by2, vy2)
        vvalid = jnp.where(lane_m & pos0, 1.0, vvalid)
        return (s_new, keep_row, vx1, vy1, vx2, vy2, vvalid)

    rowz = jnp.zeros((1, _DET), jnp.float32)
    init = (s0, jnp.full((1, _DET), -1, jnp.int32),
            rowz, rowz, rowz, rowz, rowz)
    s_fin, keep_row, vx1, vy1, vx2, vy2, vvalid = lax.fori_loop(
        0, MAX_DET, step, init)

    keep_ref[:, :] = jnp.broadcast_to(keep_row, (8, _DET))

    cx = (vx1 + vx2) * 0.5
    cy = (vy1 + vy2) * 0.5

    idx_lvl_rows = []
    w_rows = []
    for (H, W, C, scale, off) in _LEVELS:
        x = jnp.clip(cx * scale - 0.5, 0.0, W - 1.0)
        y = jnp.clip(cy * scale - 0.5, 0.0, H - 1.0)
        x0f = jnp.floor(x)
        y0f = jnp.floor(y)
        x0 = x0f.astype(jnp.int32)
        y0 = y0f.astype(jnp.int32)
        x1i = jnp.minimum(x0 + 1, W - 1)
        y1i = jnp.minimum(y0 + 1, H - 1)
        wx = x - x0f
        wy = y - y0f
        idx_lvl_rows.append(jnp.concatenate(
            [y0 * W + x0, y0 * W + x1i, y1i * W + x0, y1i * W + x1i], axis=1))
        w_rows += [(1 - wx) * (1 - wy), wx * (1 - wy), (1 - wx) * wy, wx * wy]

    idx_ref[:, :] = jnp.concatenate(
        idx_lvl_rows + [jnp.zeros((4, 4 * _DET), jnp.int32)], axis=0)
    w_ref[:, :] = jnp.concatenate(w_rows, axis=0).T
    bv_ref[:, :] = jnp.concatenate(
        [vx1, vy1, vx2, vy2, vvalid,
         jnp.zeros((3, _DET), jnp.float32)], axis=0).T


_mesh = plsc.VectorSubcoreMesh(core_axis_name="c", subcore_axis_name="s")


@functools.partial(
    pl.kernel, mesh=_mesh,
    out_type=[
        jax.ShapeDtypeStruct((4 * _DET, 256), jnp.float32),
        jax.ShapeDtypeStruct((4 * _DET, 384), jnp.float32),
        jax.ShapeDtypeStruct((4 * _DET, 640), jnp.float32),
        jax.ShapeDtypeStruct((4 * _DET, 768), jnp.float32),
    ],
    scratch_types=[
        pltpu.VMEM((16,), jnp.int32),
        pltpu.VMEM((16, 256), jnp.float32),
        pltpu.VMEM((16, 384), jnp.float32),
        pltpu.VMEM((16, 640), jnp.float32),
        pltpu.VMEM((16, 768), jnp.float32),
        pltpu.SemaphoreType.DMA,
    ],
)
def _sc_gather(idx_hbm, f1_hbm, f2_hbm, f3_hbm, f4_hbm,
               o1_hbm, o2_hbm, o3_hbm, o4_hbm,
               idx_v, r1, r2, r3, r4, sem):
    wid = lax.axis_index("s") * 2 + lax.axis_index("c")
    base = wid * 16
    for l, (fh, oh, rv) in enumerate((
            (f1_hbm, o1_hbm, r1), (f2_hbm, o2_hbm, r2),
            (f3_hbm, o3_hbm, r3), (f4_hbm, o4_hbm, r4))):
        pltpu.sync_copy(idx_hbm.at[l, pl.ds(base, 16)], idx_v)
        pltpu.async_copy(fh.at[idx_v], rv, sem).wait()
        pltpu.sync_copy(rv, oh.at[pl.ds(base, 16)])


def _mlp_body(r1_ref, r2_ref, r3_ref, r4_ref, w_ref, bv_ref,
              w1_ref, b1_ref, w2_ref, b2_ref, out_ref):
    h = jnp.zeros((_DET, 64), jnp.float32)
    rrefs = (r1_ref, r2_ref, r3_ref, r4_ref)
    for li, ((H, W, C, scale, off), rref) in enumerate(zip(_LEVELS, rrefs)):
        Cp = rref.shape[1]
        p = jnp.zeros((_DET, Cp), jnp.float32)
        for c in range(4):
            j = 4 * li + c
            p = p + w_ref[:, j:j + 1] * rref[c * _DET:(c + 1) * _DET, :]
        h = h + jnp.dot(p[:, 0:C], w1_ref[off:off + C, :],
                        preferred_element_type=jnp.float32)
    h = h + b1_ref[:, :]
    h = jnp.where(h > 0, h, 0.01 * h)
    h = jnp.dot(h, w2_ref[:, :], preferred_element_type=jnp.float32) + b2_ref[:, :]
    h = jnp.where(h > 0, h, 0.01 * h)
    out = jnp.concatenate(
        [bv_ref[:, 0:4] * (1.0 / IMG_SIZE), h], axis=1)
    out_ref[:, :] = out * bv_ref[:, 4:5]


def kernel(boxes, scores, feat1, feat2, feat3, feat4, W1, b1, W2, b2):
    pad = _NP - _N
    bt = jnp.pad(boxes.T, ((0, 0), (0, pad)))
    x1 = bt[0].reshape(_R, _C)
    y1 = bt[1].reshape(_R, _C)
    x2 = bt[2].reshape(_R, _C)
    y2 = bt[3].reshape(_R, _C)
    sp = jnp.pad(scores, (0, pad)).reshape(_R, _C)
    bsm = jnp.pad(boxes, ((0, pad), (0, 0))).reshape(-1)
    f1 = jnp.pad(feat1.reshape(192, 64 * 64).T, ((0, 0), (0, 64)))
    f2 = feat2.reshape(384, 32 * 32).T
    f3 = jnp.pad(feat3.reshape(576, 16 * 16).T, ((0, 0), (0, 64)))
    f4 = feat4.reshape(768, 8 * 8).T
    vmem = pl.BlockSpec(memory_space=pltpu.VMEM)
    keep_p, idx_sc, w_cols, bv = pl.pallas_call(
        _nms_body,
        in_specs=[vmem, vmem, vmem, vmem, vmem,
                  pl.BlockSpec(memory_space=pltpu.SMEM)],
        out_shape=[
            jax.ShapeDtypeStruct((8, _DET), jnp.int32),
            jax.ShapeDtypeStruct((8, 4 * _DET), jnp.int32),
            jax.ShapeDtypeStruct((_DET, 16), jnp.float32),
            jax.ShapeDtypeStruct((_DET, 8), jnp.float32),
        ],
    )(sp, x1, y1, x2, y2, bsm)
    r1, r2, r3, r4 = _sc_gather(idx_sc, f1, f2, f3, f4)
    out_p = pl.pallas_call(
        _mlp_body,
        out_shape=jax.ShapeDtypeStruct((_DET, 68), jnp.float32),
    )(r1, r2, r3, r4, w_cols, bv, W1, b1.reshape(1, 64),
      W2, b2.reshape(1, 64))
    return out_p[:MAX_DET], keep_p[0, :MAX_DET]


# speedup vs baseline: 1.7378x; 1.7378x over previous
import jax
import jax.numpy as jnp
from jax import lax
from jax.experimental import pallas as pl
from jax.experimental.pallas import tpu as pltpu

CONF_THRES = 0.1
IOU_THRES = 0.6
MAX_DET = 100
IMG_SIZE = 512.0

_N = 5000
_NP = 5120
_R, _C = 8, 640
_DET = 128

_LEVELS = (
    (64, 64, 192, 1.0 / 8, 0),
    (32, 32, 384, 1.0 / 16, 192),
    (16, 16, 576, 1.0 / 32, 576),
    (8, 8, 768, 1.0 / 64, 1152),
)


def _body(sc_ref, x1_ref, y1_ref, x2_ref, y2_ref, bsm_ref,
          f1_ref, f2_ref, f3_ref, f4_ref,
          w1_ref, b1_ref, w2_ref, b2_ref,
          out_ref, keep_ref):
    s0 = jnp.where(sc_ref[:, :] > CONF_THRES, sc_ref[:, :], -1.0)
    lane_iota = lax.broadcasted_iota(jnp.int32, (1, _DET), 1)

    def step(t, carry):
        s, keep_row, vx1, vy1, vx2, vy2, vvalid = carry
        x1 = x1_ref[:, :]
        y1 = y1_ref[:, :]
        x2 = x2_ref[:, :]
        y2 = y2_ref[:, :]
        ridx = lax.broadcasted_iota(jnp.int32, (_R, _C), 0)
        cidx = lax.broadcasted_iota(jnp.int32, (_R, _C), 1)
        gidx = ridx * _C + cidx
        rowmax = jnp.max(s, axis=1, keepdims=True)
        rowarg = jnp.argmax(s, axis=1, keepdims=True)
        riota = lax.broadcasted_iota(jnp.int32, (_R, 1), 0)
        best = jnp.max(rowmax, axis=0, keepdims=True)
        rhit = rowmax == best
        r = jnp.min(jnp.where(rhit, riota, _R), axis=0, keepdims=True)
        carg = jnp.min(jnp.where(riota == r, rowarg, _C),
                       axis=0, keepdims=True)
        idxv = r * _C + carg
        idx_s = idxv[0, 0]
        base = idx_s * 4
        bx1 = bsm_ref[base]
        by1 = bsm_ref[base + 1]
        bx2 = bsm_ref[base + 2]
        by2 = bsm_ref[base + 3]
        barea = (bx2 - bx1) * (by2 - by1)
        area = (x2 - x1) * (y2 - y1)
        xx1 = jnp.maximum(bx1, x1)
        yy1 = jnp.maximum(by1, y1)
        xx2 = jnp.minimum(bx2, x2)
        yy2 = jnp.minimum(by2, y2)
        inter = jnp.maximum(xx2 - xx1, 0.0) * jnp.maximum(yy2 - yy1, 0.0)
        iou = inter / (barea + area - inter + 1e-9)
        pos = best > 0.0
        suppress = ((iou > IOU_THRES) | (gidx == idx_s)) & pos
        s_new = jnp.where(suppress, -1.0, s)
        pos0 = pos[0:1, 0:1]
        out_idx = jnp.where(pos0, idx_s, -1)
        lane_m = lane_iota == t
        keep_row = jnp.where(lane_m, out_idx, keep_row)
        vx1 = jnp.where(lane_m, bx1, vx1)
        vy1 = jnp.where(lane_m, by1, vy1)
        vx2 = jnp.where(lane_m, bx2, vx2)
        vy2 = jnp.where(lane_m, by2, vy2)
        vvalid = jnp.where(lane_m & pos0, 1.0, vvalid)
        return (s_new, keep_row, vx1, vy1, vx2, vy2, vvalid)

    rowz = jnp.zeros((1, _DET), jnp.float32)
    init = (s0, jnp.full((1, _DET), -1, jnp.int32),
            rowz, rowz, rowz, rowz, rowz)
    s_fin, keep_row, vx1, vy1, vx2, vy2, vvalid = lax.fori_loop(
        0, MAX_DET, step, init, unroll=2)

    keep_ref[:, :] = jnp.broadcast_to(keep_row, (8, _DET))

    cx = (vx1 + vx2) * 0.5
    cy = (vy1 + vy2) * 0.5

    idx_rows = []
    w_rows = []
    for (H, W, C, scale, off) in _LEVELS:
        x = jnp.clip(cx * scale - 0.5, 0.0, W - 1.0)
        y = jnp.clip(cy * scale - 0.5, 0.0, H - 1.0)
        x0f = jnp.floor(x)
        y0f = jnp.floor(y)
        x0 = x0f.astype(jnp.int32)
        y0 = y0f.astype(jnp.int32)
        x1i = jnp.minimum(x0 + 1, W - 1)
        y1i = jnp.minimum(y0 + 1, H - 1)
        wx = x - x0f
        wy = y - y0f
        idx_rows += [y0 * W + x0, y0 * W + x1i, y1i * W + x0, y1i * W + x1i]
        w_rows += [(1 - wx) * (1 - wy), wx * (1 - wy), (1 - wx) * wy, wx * wy]

    idx_cols = jnp.concatenate(idx_rows + idx_rows[:0], axis=0).T
    w_cols = jnp.concatenate(w_rows, axis=0).T
    bbox_valid = jnp.concatenate(
        [vx1, vy1, vx2, vy2, vvalid,
         jnp.zeros((3, _DET), jnp.float32)], axis=0).T

    h = jnp.zeros((_DET, 64), jnp.float32)
    frefs = (f1_ref, f2_ref, f3_ref, f4_ref)
    for li, ((H, W, C, scale, off), fref) in enumerate(zip(_LEVELS, frefs)):
        hw_iota = lax.broadcasted_iota(jnp.int32, (_DET, H * W), 1)
        m = jnp.zeros((_DET, H * W), jnp.float32)
        for c in range(4):
            j = 4 * li + c
            m = m + ((hw_iota == idx_cols[:, j:j + 1]).astype(jnp.float32)
                     * w_cols[:, j:j + 1])
        p = jnp.dot(m, fref[:, :], preferred_element_type=jnp.float32)
        h = h + jnp.dot(p, w1_ref[off:off + C, :],
                        preferred_element_type=jnp.float32)
    h = h + b1_ref[:, :]
    h = jnp.where(h > 0, h, 0.01 * h)
    h = jnp.dot(h, w2_ref[:, :], preferred_element_type=jnp.float32) + b2_ref[:, :]
    h = jnp.where(h > 0, h, 0.01 * h)

    out = jnp.concatenate(
        [bbox_valid[:, 0:4] * (1.0 / IMG_SIZE), h], axis=1)
    out_ref[:, :] = out * bbox_valid[:, 4:5]


def kernel(boxes, scores, feat1, feat2, feat3, feat4, W1, b1, W2, b2):
    pad = _NP - _N
    bt = jnp.pad(boxes.T, ((0, 0), (0, pad)))
    x1 = bt[0].reshape(_R, _C)
    y1 = bt[1].reshape(_R, _C)
    x2 = bt[2].reshape(_R, _C)
    y2 = bt[3].reshape(_R, _C)
    sp = jnp.pad(scores, (0, pad)).reshape(_R, _C)
    bsm = jnp.pad(boxes, ((0, pad), (0, 0))).reshape(-1)
    f1 = feat1.reshape(192, 64 * 64).T
    f2 = feat2.reshape(384, 32 * 32).T
    f3 = feat3.reshape(576, 16 * 16).T
    f4 = feat4.reshape(768, 8 * 8).T
    vmem = pl.BlockSpec(memory_space=pltpu.VMEM)
    out_p, keep_p = pl.pallas_call(
        _body,
        in_specs=[vmem, vmem, vmem, vmem, vmem,
                  pl.BlockSpec(memory_space=pltpu.SMEM),
                  vmem, vmem, vmem, vmem, vmem, vmem, vmem, vmem],
        out_shape=[
            jax.ShapeDtypeStruct((_DET, 68), jnp.float32),
            jax.ShapeDtypeStruct((8, _DET), jnp.int32),
        ],
    )(sp, x1, y1, x2, y2, bsm, f1, f2, f3, f4,
      W1, b1.reshape(1, 64), W2, b2.reshape(1, 64))
    return out_p[:MAX_DET], keep_p[0, :MAX_DET]


# speedup vs baseline: 1.7682x; 1.0175x over previous
import jax
import jax.numpy as jnp
from jax import lax
from jax.experimental import pallas as pl
from jax.experimental.pallas import tpu as pltpu

CONF_THRES = 0.1
IOU_THRES = 0.6
MAX_DET = 100
IMG_SIZE = 512.0

_N = 5000
_NP = 5120
_R, _C = 8, 640
_DET = 128

_LEVELS = (
    (64, 64, 192, 1.0 / 8, 0),
    (32, 32, 384, 1.0 / 16, 192),
    (16, 16, 576, 1.0 / 32, 576),
    (8, 8, 768, 1.0 / 64, 1152),
)


def _body(sc_ref, x1_ref, y1_ref, x2_ref, y2_ref, bsm_ref,
          f1_ref, f2_ref, f3_ref, f4_ref,
          w1_ref, b1_ref, w2_ref, b2_ref,
          out_ref, keep_ref):
    s0 = jnp.where(sc_ref[:, :] > CONF_THRES, sc_ref[:, :], -1.0)
    lane_iota = lax.broadcasted_iota(jnp.int32, (1, _DET), 1)

    def step(t, carry):
        s, keep_row, vx1, vy1, vx2, vy2, vvalid = carry
        x1 = x1_ref[:, :]
        y1 = y1_ref[:, :]
        x2 = x2_ref[:, :]
        y2 = y2_ref[:, :]
        ridx = lax.broadcasted_iota(jnp.int32, (_R, _C), 0)
        cidx = lax.broadcasted_iota(jnp.int32, (_R, _C), 1)
        gidx = ridx * _C + cidx
        rowmax = jnp.max(s, axis=1, keepdims=True)
        rowarg = jnp.argmax(s, axis=1, keepdims=True)
        riota = lax.broadcasted_iota(jnp.int32, (_R, 1), 0)
        best = jnp.max(rowmax, axis=0, keepdims=True)
        rhit = rowmax == best
        r = jnp.min(jnp.where(rhit, riota, _R), axis=0, keepdims=True)
        carg = jnp.min(jnp.where(riota == r, rowarg, _C),
                       axis=0, keepdims=True)
        idxv = r * _C + carg
        idx_s = idxv[0, 0]
        base = idx_s * 4
        bx1 = bsm_ref[base]
        by1 = bsm_ref[base + 1]
        bx2 = bsm_ref[base + 2]
        by2 = bsm_ref[base + 3]
        barea = (bx2 - bx1) * (by2 - by1)
        area = (x2 - x1) * (y2 - y1)
        xx1 = jnp.maximum(bx1, x1)
        yy1 = jnp.maximum(by1, y1)
        xx2 = jnp.minimum(bx2, x2)
        yy2 = jnp.minimum(by2, y2)
        inter = jnp.maximum(xx2 - xx1, 0.0) * jnp.maximum(yy2 - yy1, 0.0)
        iou = inter / (barea + area - inter + 1e-9)
        pos = best > 0.0
        suppress = ((iou > IOU_THRES) | (gidx == idx_s)) & pos
        s_new = jnp.where(suppress, -1.0, s)
        pos0 = pos[0:1, 0:1]
        out_idx = jnp.where(pos0, idx_s, -1)
        lane_m = lane_iota == t
        keep_row = jnp.where(lane_m, out_idx, keep_row)
        vx1 = jnp.where(lane_m, bx1, vx1)
        vy1 = jnp.where(lane_m, by1, vy1)
        vx2 = jnp.where(lane_m, bx2, vx2)
        vy2 = jnp.where(lane_m, by2, vy2)
        vvalid = jnp.where(lane_m & pos0, 1.0, vvalid)
        return (s_new, keep_row, vx1, vy1, vx2, vy2, vvalid)

    rowz = jnp.zeros((1, _DET), jnp.float32)
    init = (s0, jnp.full((1, _DET), -1, jnp.int32),
            rowz, rowz, rowz, rowz, rowz)
    s_fin, keep_row, vx1, vy1, vx2, vy2, vvalid = lax.fori_loop(
        0, MAX_DET, step, init, unroll=4)

    keep_ref[:, :] = jnp.broadcast_to(keep_row, (8, _DET))

    cx = (vx1 + vx2) * 0.5
    cy = (vy1 + vy2) * 0.5

    idx_rows = []
    w_rows = []
    for (H, W, C, scale, off) in _LEVELS:
        x = jnp.clip(cx * scale - 0.5, 0.0, W - 1.0)
        y = jnp.clip(cy * scale - 0.5, 0.0, H - 1.0)
        x0f = jnp.floor(x)
        y0f = jnp.floor(y)
        x0 = x0f.astype(jnp.int32)
        y0 = y0f.astype(jnp.int32)
        x1i = jnp.minimum(x0 + 1, W - 1)
        y1i = jnp.minimum(y0 + 1, H - 1)
        wx = x - x0f
        wy = y - y0f
        idx_rows += [y0 * W + x0, y0 * W + x1i, y1i * W + x0, y1i * W + x1i]
        w_rows += [(1 - wx) * (1 - wy), wx * (1 - wy), (1 - wx) * wy, wx * wy]

    idx_cols = jnp.concatenate(idx_rows + idx_rows[:0], axis=0).T
    w_cols = jnp.concatenate(w_rows, axis=0).T
    bbox_valid = jnp.concatenate(
        [vx1, vy1, vx2, vy2, vvalid,
         jnp.zeros((3, _DET), jnp.float32)], axis=0).T

    h = jnp.zeros((_DET, 64), jnp.float32)
    frefs = (f1_ref, f2_ref, f3_ref, f4_ref)
    for li, ((H, W, C, scale, off), fref) in enumerate(zip(_LEVELS, frefs)):
        hw_iota = lax.broadcasted_iota(jnp.int32, (_DET, H * W), 1)
        m = jnp.zeros((_DET, H * W), jnp.float32)
        for c in range(4):
            j = 4 * li + c
            m = m + ((hw_iota == idx_cols[:, j:j + 1]).astype(jnp.float32)
                     * w_cols[:, j:j + 1])
        p = jnp.dot(m, fref[:, :], preferred_element_type=jnp.float32)
        h = h + jnp.dot(p, w1_ref[off:off + C, :],
                        preferred_element_type=jnp.float32)
    h = h + b1_ref[:, :]
    h = jnp.where(h > 0, h, 0.01 * h)
    h = jnp.dot(h, w2_ref[:, :], preferred_element_type=jnp.float32) + b2_ref[:, :]
    h = jnp.where(h > 0, h, 0.01 * h)

    out = jnp.concatenate(
        [bbox_valid[:, 0:4] * (1.0 / IMG_SIZE), h], axis=1)
    out_ref[:, :] = out * bbox_valid[:, 4:5]


def kernel(boxes, scores, feat1, feat2, feat3, feat4, W1, b1, W2, b2):
    pad = _NP - _N
    bt = jnp.pad(boxes.T, ((0, 0), (0, pad)))
    x1 = bt[0].reshape(_R, _C)
    y1 = bt[1].reshape(_R, _C)
    x2 = bt[2].reshape(_R, _C)
    y2 = bt[3].reshape(_R, _C)
    sp = jnp.pad(scores, (0, pad)).reshape(_R, _C)
    bsm = jnp.pad(boxes, ((0, pad), (0, 0))).reshape(-1)
    f1 = feat1.reshape(192, 64 * 64).T
    f2 = feat2.reshape(384, 32 * 32).T
    f3 = feat3.reshape(576, 16 * 16).T
    f4 = feat4.reshape(768, 8 * 8).T
    vmem = pl.BlockSpec(memory_space=pltpu.VMEM)
    out_p, keep_p = pl.pallas_call(
        _body,
        in_specs=[vmem, vmem, vmem, vmem, vmem,
                  pl.BlockSpec(memory_space=pltpu.SMEM),
                  vmem, vmem, vmem, vmem, vmem, vmem, vmem, vmem],
        out_shape=[
            jax.ShapeDtypeStruct((_DET, 68), jnp.float32),
            jax.ShapeDtypeStruct((8, _DET), jnp.int32),
        ],
    )(sp, x1, y1, x2, y2, bsm, f1, f2, f3, f4,
      W1, b1.reshape(1, 64), W2, b2.reshape(1, 64))
    return out_p[:MAX_DET], keep_p[0, :MAX_DET]
